# baseline (speedup 1.0000x reference)
import functools

import jax
import jax.numpy as jnp
from jax import lax
from jax.experimental import pallas as pl
from jax.experimental.pallas import tpu as pltpu, tpu_sc as plsc

VOCAB = 100000
MAX_LEN = 200
DIM = 32
BATCH = 4096

NC, NS, L = 2, 16, 16
NW = NC * NS
NCALLS = 8
BC = BATCH // NCALLS
SEQ_PER_W = BC // NW
SEQ_PER_GROUP = 4
GROUP = SEQ_PER_GROUP * MAX_LEN
NGROUPS = SEQ_PER_W // SEQ_PER_GROUP
GCHUNK = 80
NBUF = 4
PRIME = NBUF - 1


def _make_kernel():
  mesh = plsc.VectorSubcoreMesh(
      core_axis_name="c", subcore_axis_name="s", num_cores=NC, num_subcores=NS
  )

  @functools.partial(
      pl.kernel,
      mesh=mesh,
      compiler_params=pltpu.CompilerParams(use_tc_tiling_on_sc=False),
      out_type=jax.ShapeDtypeStruct((BC, MAX_LEN, 128), jnp.float32),
      scratch_types=[
          [pltpu.VMEM((SEQ_PER_GROUP, MAX_LEN), jnp.int32) for _ in range(NBUF)],
          [pltpu.VMEM((SEQ_PER_GROUP, MAX_LEN, DIM), jnp.float32)
           for _ in range(NBUF)],
          pltpu.VMEM((MAX_LEN, DIM), jnp.float32),
          [pltpu.SemaphoreType.DMA for _ in range(NBUF)],
          [pltpu.SemaphoreType.DMA for _ in range(NBUF)],
      ],
  )
  def k(x_hbm, table_hbm, pos_hbm, out_hbm, idx_v, rows_v, pos_v, gsem, wsem):
    wid = lax.axis_index("s") * NC + lax.axis_index("c")
    pltpu.sync_copy(pos_hbm, pos_v)
    w_seq = wid * SEQ_PER_W

    def fire_group(g, b):
      seq0 = w_seq + g * SEQ_PER_GROUP
      pltpu.sync_copy(x_hbm.at[pl.ds(seq0, SEQ_PER_GROUP)], idx_v[b])
      for s in range(SEQ_PER_GROUP):
        off = 0
        for c in (GCHUNK, GCHUNK, MAX_LEN - 2 * GCHUNK):
          pltpu.async_copy(
              table_hbm.at[idx_v[b].at[s].at[pl.ds(off, c)]],
              rows_v[b].at[s].at[pl.ds(off, c)],
              gsem[b],
          )
          off += c

    def wait_gathers(b):
      pltpu.make_async_copy(
          out_hbm.at[pl.ds(0, SEQ_PER_GROUP), :, pl.ds(0, DIM)],
          rows_v[b], gsem[b],
      ).wait()

    def wait_write(b):
      pltpu.make_async_copy(
          rows_v[b], out_hbm.at[pl.ds(0, SEQ_PER_GROUP), :, pl.ds(0, DIM)],
          wsem[b],
      ).wait()

    for p in range(PRIME):
      fire_group(p, p)

    def h_body(h, carry):
      for b in range(NBUF):
        g = h * NBUF + b
        bf = (b + PRIME) % NBUF
        gf = g + PRIME

        @pl.when(jnp.logical_and(gf < NGROUPS, g >= 1))
        def _():
          wait_write(bf)
          fire_group(gf, bf)

        @pl.when(jnp.logical_and(gf < NGROUPS, g < 1))
        def _():
          fire_group(gf, bf)

        wait_gathers(b)

        def add_body(l, c):
          p0 = pos_v[l, pl.ds(0, L)]
          p1 = pos_v[l, pl.ds(L, L)]
          for s in range(SEQ_PER_GROUP):
            rows_v[b][s, l, pl.ds(0, L)] = rows_v[b][s, l, pl.ds(0, L)] + p0
            rows_v[b][s, l, pl.ds(L, L)] = rows_v[b][s, l, pl.ds(L, L)] + p1
          return c
        lax.fori_loop(0, MAX_LEN, add_body, 0)

        pltpu.async_copy(
            rows_v[b],
            out_hbm.at[pl.ds(w_seq + g * SEQ_PER_GROUP, SEQ_PER_GROUP),
                       :, pl.ds(0, DIM)],
            wsem[b],
        )
      return carry

    lax.fori_loop(0, NGROUPS // NBUF, h_body, 0)

    for b in range(NBUF):
      wait_write(b)

  return k


_kernel_cache = []


def kernel(x, token_table, pos_table):
  if not _kernel_cache:
    _kernel_cache.append(_make_kernel())
  k = _kernel_cache[0]
  xi = x.astype(jnp.int32)
  parts = [
      k(xi[i * BC:(i + 1) * BC], token_table, pos_table)
      for i in range(NCALLS)
  ]
  return jnp.concatenate(parts, axis=0)[:, :, :DIM]


# speedup vs baseline: 1.8120x; 1.8120x over previous
import functools

import jax
import jax.numpy as jnp
from jax import lax
from jax.experimental import pallas as pl
from jax.experimental.pallas import tpu as pltpu, tpu_sc as plsc

VOCAB = 100000
MAX_LEN = 200
DIM = 32
BATCH = 4096

NC, NS, L = 2, 16, 16
NW = NC * NS
NCALLS = 1
BC = BATCH // NCALLS
SEQ_PER_W = BC // NW
SEQ_PER_GROUP = 4
GROUP = SEQ_PER_GROUP * MAX_LEN
NGROUPS = SEQ_PER_W // SEQ_PER_GROUP
GCHUNK = 80
NBUF = 4
PRIME = NBUF - 1


def _make_kernel():
  mesh = plsc.VectorSubcoreMesh(
      core_axis_name="c", subcore_axis_name="s", num_cores=NC, num_subcores=NS
  )

  @functools.partial(
      pl.kernel,
      mesh=mesh,
      compiler_params=pltpu.CompilerParams(use_tc_tiling_on_sc=False),
      out_type=jax.ShapeDtypeStruct((BC, MAX_LEN, 128), jnp.float32),
      scratch_types=[
          [pltpu.VMEM((SEQ_PER_GROUP, MAX_LEN), jnp.int32) for _ in range(NBUF)],
          [pltpu.VMEM((SEQ_PER_GROUP, MAX_LEN, DIM), jnp.float32)
           for _ in range(NBUF)],
          pltpu.VMEM((MAX_LEN, DIM), jnp.float32),
          [pltpu.SemaphoreType.DMA for _ in range(NBUF)],
          [pltpu.SemaphoreType.DMA for _ in range(NBUF)],
      ],
  )
  def k(x_hbm, table_hbm, pos_hbm, out_hbm, idx_v, rows_v, pos_v, gsem, wsem):
    wid = lax.axis_index("s") * NC + lax.axis_index("c")
    pltpu.sync_copy(pos_hbm, pos_v)
    w_seq = wid * SEQ_PER_W

    def fire_group(g, b):
      seq0 = w_seq + g * SEQ_PER_GROUP
      pltpu.sync_copy(x_hbm.at[pl.ds(seq0, SEQ_PER_GROUP)], idx_v[b])
      for s in range(SEQ_PER_GROUP):
        off = 0
        for c in (GCHUNK, GCHUNK, MAX_LEN - 2 * GCHUNK):
          pltpu.async_copy(
              table_hbm.at[idx_v[b].at[s].at[pl.ds(off, c)]],
              rows_v[b].at[s].at[pl.ds(off, c)],
              gsem[b],
          )
          off += c

    def wait_gathers(b):
      pltpu.make_async_copy(
          out_hbm.at[pl.ds(0, SEQ_PER_GROUP), :, pl.ds(0, DIM)],
          rows_v[b], gsem[b],
      ).wait()

    def wait_write(b):
      pltpu.make_async_copy(
          rows_v[b], out_hbm.at[pl.ds(0, SEQ_PER_GROUP), :, pl.ds(0, DIM)],
          wsem[b],
      ).wait()

    for p in range(PRIME):
      fire_group(p, p)

    def h_body(h, carry):
      for b in range(NBUF):
        g = h * NBUF + b
        bf = (b + PRIME) % NBUF
        gf = g + PRIME

        @pl.when(jnp.logical_and(gf < NGROUPS, g >= 1))
        def _():
          wait_write(bf)
          fire_group(gf, bf)

        @pl.when(jnp.logical_and(gf < NGROUPS, g < 1))
        def _():
          fire_group(gf, bf)

        wait_gathers(b)

        def add_body(l, c):
          p0 = pos_v[l, pl.ds(0, L)]
          p1 = pos_v[l, pl.ds(L, L)]
          for s in range(SEQ_PER_GROUP):
            rows_v[b][s, l, pl.ds(0, L)] = rows_v[b][s, l, pl.ds(0, L)] + p0
            rows_v[b][s, l, pl.ds(L, L)] = rows_v[b][s, l, pl.ds(L, L)] + p1
          return c
        lax.fori_loop(0, MAX_LEN, add_body, 0)

        pltpu.async_copy(
            rows_v[b],
            out_hbm.at[pl.ds(w_seq + g * SEQ_PER_GROUP, SEQ_PER_GROUP),
                       :, pl.ds(0, DIM)],
            wsem[b],
        )
      return carry

    lax.fori_loop(0, NGROUPS // NBUF, h_body, 0)

    for b in range(NBUF):
      wait_write(b)

  return k


_kernel_cache = []


def kernel(x, token_table, pos_table):
  if not _kernel_cache:
    _kernel_cache.append(_make_kernel())
  k = _kernel_cache[0]
  xi = x.astype(jnp.int32)
  parts = [
      k(xi[i * BC:(i + 1) * BC], token_table, pos_table)
      for i in range(NCALLS)
  ]
  return jnp.concatenate(parts, axis=0)[:, :, :DIM]
